# Initial kernel scaffold; baseline (speedup 1.0000x reference)
import jax
import jax.numpy as jnp
from jax.experimental import pallas as pl

D_MODEL = 768
N_HEADS = 12
HEAD_DIM = 64
N_CLUSTERS = 32
CLUSTER_K = 64
N_ROPE = 32
N_TOK = 2048

_PREC = jax.lax.Precision.HIGHEST
_LIM = 0x3FA00000


def _hi_lo(x):
    hi = x.astype(jnp.bfloat16)
    lo = (x - hi.astype(jnp.float32)).astype(jnp.bfloat16)
    return hi, lo


def _dot(a, b, dims):
    return jax.lax.dot_general(a, b, (dims, ((), ())),
                               preferred_element_type=jnp.float32)


def _proj_kernel(x_ref, pos_ref, wq_ref, wk_ref, wv_ref, wr_ref, rmsw_ref,
                 dirs_ref, freqs_ref, cin_ref, cout_ref,
                 q_ref, k_ref, v_ref, score_ref):
    x = x_ref[0]
    pos = pos_ref[0]
    xs = x * jax.lax.rsqrt(jnp.mean(x * x, axis=-1, keepdims=True) + 1e-6)
    xs = xs * rmsw_ref[0:1, :]
    proj = (pos[:, 0:1] * dirs_ref[0:1, :] + pos[:, 1:2] * dirs_ref[1:2, :]
            + pos[:, 2:3] * dirs_ref[2:3, :])
    angles = proj * freqs_ref[0:1, :]
    emb = jnp.concatenate([jnp.sin(angles), jnp.cos(angles)], axis=1)
    rope = jnp.dot(emb, wr_ref[...], precision=_PREC,
                   preferred_element_type=jnp.float32)
    q = jnp.dot(xs, wq_ref[...], precision=_PREC,
                preferred_element_type=jnp.float32) + rope
    k = jnp.dot(xs, wk_ref[...], precision=_PREC,
                preferred_element_type=jnp.float32) + rope
    v = jnp.dot(xs, wv_ref[...], precision=_PREC,
                preferred_element_type=jnp.float32)
    q_ref[0] = q
    k_ref[0] = k
    v_ref[0] = v

    def _norm(c):
        return c / jnp.sqrt(jnp.clip(jnp.sum(c * c, axis=-1, keepdims=True),
                                     1e-12, None))
    cin = _norm(cin_ref[...])
    cout = _norm(cout_ref[...])
    p_norm2 = jnp.sum(pos * pos, axis=-1, keepdims=True)
    for h in range(N_HEADS):
        qh = q[:, h * HEAD_DIM:(h + 1) * HEAD_DIM]
        kh = k[:, h * HEAD_DIM:(h + 1) * HEAD_DIM]
        cin_h = cin[h * N_CLUSTERS:(h + 1) * N_CLUSTERS, :]
        cout_h = cout[h * N_CLUSTERS:(h + 1) * N_CLUSTERS, :]
        cn_q = jnp.sqrt(jnp.clip(jnp.sum(qh * qh, axis=-1, keepdims=True)
                                 + p_norm2, 1e-12, None))
        cn_k = jnp.sqrt(jnp.clip(jnp.sum(kh * kh, axis=-1, keepdims=True)
                                 + p_norm2, 1e-12, None))
        in_s = (jax.lax.dot_general(qh, cin_h[:, :HEAD_DIM],
                                    (((1,), (1,)), ((), ())),
                                    precision=_PREC,
                                    preferred_element_type=jnp.float32)
                + jax.lax.dot_general(pos, cin_h[:, HEAD_DIM:HEAD_DIM + 3],
                                      (((1,), (1,)), ((), ())),
                                      precision=_PREC,
                                      preferred_element_type=jnp.float32))
        out_s = (jax.lax.dot_general(kh, cout_h[:, :HEAD_DIM],
                                     (((1,), (1,)), ((), ())),
                                     precision=_PREC,
                                     preferred_element_type=jnp.float32)
                 + jax.lax.dot_general(pos, cout_h[:, HEAD_DIM:HEAD_DIM + 3],
                                       (((1,), (1,)), ((), ())),
                                       precision=_PREC,
                                       preferred_element_type=jnp.float32))
        score_ref[0, :, h * N_CLUSTERS:(h + 1) * N_CLUSTERS] = in_s / cn_q
        score_ref[0, :, 384 + h * N_CLUSTERS:384 + (h + 1) * N_CLUSTERS] = \
            out_s / cn_k


def _topk_kernel(score_ref, mask_ref, rank_ref):
    s = score_ref[0]
    b = jax.lax.bitcast_convert_type(s, jnp.int32)
    neg = b < 0
    m = jnp.where(neg, jnp.bitwise_xor(jnp.bitwise_not(b),
                                       jnp.int32(-2147483648)), b)
    ncols = s.shape[1]
    lo = jnp.full((1, ncols), -_LIM - 1, dtype=jnp.int32)
    hi = jnp.full((1, ncols), _LIM, dtype=jnp.int32)
    for _ in range(31):
        mid = lo + (hi - lo + 1) // 2
        cnt = jnp.sum((m >= mid).astype(jnp.int32), axis=0, keepdims=True)
        ge = cnt >= CLUSTER_K
        lo = jnp.where(ge, mid, lo)
        hi = jnp.where(ge, hi, mid - 1)
    vstar = lo
    gt = m > vstar
    eq = m == vstar
    cnt_gt = jnp.sum(gt.astype(jnp.int32), axis=0, keepdims=True)
    r = CLUSTER_K - cnt_gt
    idx = jax.lax.broadcasted_iota(jnp.int32, m.shape, 0)
    lo2 = jnp.zeros((1, ncols), dtype=jnp.int32)
    hi2 = jnp.full((1, ncols), N_TOK - 1, dtype=jnp.int32)
    for _ in range(11):
        mid2 = (lo2 + hi2) // 2
        c2 = jnp.sum((eq & (idx <= mid2)).astype(jnp.int32),
                     axis=0, keepdims=True)
        geq = c2 >= r
        hi2 = jnp.where(geq, mid2, hi2)
        lo2 = jnp.where(geq, lo2, mid2 + 1)
    sel = gt | (eq & (idx <= hi2))
    maskf = sel.astype(jnp.float32)
    ltri = (jax.lax.broadcasted_iota(jnp.int32, (N_TOK, N_TOK), 1)
            < jax.lax.broadcasted_iota(jnp.int32, (N_TOK, N_TOK), 0)
            ).astype(jnp.bfloat16)
    rank = jnp.dot(ltri, maskf.astype(jnp.bfloat16),
                   preferred_element_type=jnp.float32)
    mask_ref[0] = maskf
    rank_ref[0] = rank


def _attn_kernel(q_ref, k_ref, v_ref, mq_ref, rq_ref, mkv_ref, rkv_ref,
                 null_ref, o_ref):
    mq = mq_ref[0]
    rq = rq_ref[0]
    mkv = mkv_ref[0]
    rkv = rkv_ref[0]
    iota64 = jax.lax.broadcasted_iota(jnp.float32, (1, CLUSTER_K), 1)

    def build_pt(mask, rank):
        blocks = []
        for c in range(N_CLUSTERS):
            mc = mask[:, c:c + 1]
            rc = rank[:, c:c + 1]
            blocks.append(((rc == iota64) & (mc > 0.5)).astype(jnp.bfloat16))
        return jnp.concatenate(blocks, axis=1)

    ptq = build_pt(mq, rq)
    ptkv = build_pt(mkv, rkv)

    def gather(pt, xh):
        hi, lo = _hi_lo(xh)
        return (_dot(pt, hi, ((0,), (0,))) + _dot(pt, lo, ((0,), (0,))))

    q_c = gather(ptq, q_ref[0])
    k_c = gather(ptkv, k_ref[0])
    v_c = gather(ptkv, v_ref[0])

    qh_, ql_ = _hi_lo(q_c)
    kh_, kl_ = _hi_lo(k_c)
    lg = (_dot(qh_, kh_, ((1,), (1,))) + _dot(qh_, kl_, ((1,), (1,)))
          + _dot(ql_, kh_, ((1,), (1,)))) * (1.0 / 8.0)
    null = null_ref[0:1, :]
    nh_, nl_ = _hi_lo(null)
    ln = (_dot(qh_, nh_, ((1,), (1,))) + _dot(qh_, nl_, ((1,), (1,)))
          + _dot(ql_, nh_, ((1,), (1,)))) * (1.0 / 8.0)

    dblocks = [lg[c * CLUSTER_K:(c + 1) * CLUSTER_K,
                  c * CLUSTER_K:(c + 1) * CLUSTER_K]
               for c in range(N_CLUSTERS)]
    lgc = jnp.concatenate(dblocks, axis=0)
    mx = jnp.maximum(jnp.max(lgc, axis=1, keepdims=True), ln)
    e = jnp.exp(lgc - mx)
    en = jnp.exp(ln - mx)
    denom = jnp.sum(e, axis=1, keepdims=True) + en
    attnc = e / denom
    attn_null = en / denom

    ah, al = _hi_lo(attnc)

    def assemble(blockmat):
        rows = []
        for c in range(N_CLUSTERS):
            blk = blockmat[c * CLUSTER_K:(c + 1) * CLUSTER_K, :]
            pieces = []
            if c > 0:
                pieces.append(jnp.zeros((CLUSTER_K, c * CLUSTER_K),
                                        dtype=blk.dtype))
            pieces.append(blk)
            if c < N_CLUSTERS - 1:
                pieces.append(jnp.zeros(
                    (CLUSTER_K, (N_CLUSTERS - 1 - c) * CLUSTER_K),
                    dtype=blk.dtype))
            rows.append(jnp.concatenate(pieces, axis=1)
                        if len(pieces) > 1 else pieces[0])
        return jnp.concatenate(rows, axis=0)

    afh = assemble(ah)
    afl = assemble(al)
    vh_, vl_ = _hi_lo(v_c)
    out_c = (_dot(afh, vh_, ((1,), (0,))) + _dot(afh, vl_, ((1,), (0,)))
             + _dot(afl, vh_, ((1,), (0,))))
    out_c = out_c + attn_null * null
    och, ocl = _hi_lo(out_c)
    acc = _dot(ptq, och, ((1,), (0,))) + _dot(ptq, ocl, ((1,), (0,)))
    cnt = jnp.sum(mq, axis=1, keepdims=True)
    o_ref[0] = acc / jnp.maximum(cnt, 1.0)


def _wo_kernel(x_ref, wo_ref, y_ref):
    y_ref[0] = jnp.dot(x_ref[0], wo_ref[...], precision=_PREC,
                       preferred_element_type=jnp.float32)


@jax.jit
def kernel(x, unit_point_positions, Wq, Wk, Wv, Wo, W_rope, rms_w, null_vec,
           input_centroids, output_centroids, rope_dirs, rope_freqs):
    S = x.shape[0]
    f32 = jnp.float32
    pos = unit_point_positions
    cinP = input_centroids.reshape(N_HEADS * N_CLUSTERS, HEAD_DIM + 3)
    coutP = output_centroids.reshape(N_HEADS * N_CLUSTERS, HEAD_DIM + 3)
    dirsT = rope_dirs.T
    freqs = rope_freqs.reshape(1, N_ROPE)
    rmsw = rms_w.reshape(1, D_MODEL)
    null_h = null_vec.reshape(N_HEADS, HEAD_DIM)

    def full(shape):
        return pl.BlockSpec(shape, lambda s, _n=len(shape): (0,) * _n)

    q, k, v, score = pl.pallas_call(
        _proj_kernel,
        grid=(S,),
        in_specs=[
            pl.BlockSpec((1, N_TOK, D_MODEL), lambda s: (s, 0, 0)),
            pl.BlockSpec((1, N_TOK, 3), lambda s: (s, 0, 0)),
            full((D_MODEL, D_MODEL)), full((D_MODEL, D_MODEL)),
            full((D_MODEL, D_MODEL)), full((2 * N_ROPE, D_MODEL)),
            full((1, D_MODEL)), full((3, N_ROPE)), full((1, N_ROPE)),
            full((N_HEADS * N_CLUSTERS, HEAD_DIM + 3)),
            full((N_HEADS * N_CLUSTERS, HEAD_DIM + 3)),
        ],
        out_specs=[
            pl.BlockSpec((1, N_TOK, D_MODEL), lambda s: (s, 0, 0)),
            pl.BlockSpec((1, N_TOK, D_MODEL), lambda s: (s, 0, 0)),
            pl.BlockSpec((1, N_TOK, D_MODEL), lambda s: (s, 0, 0)),
            pl.BlockSpec((1, N_TOK, 768), lambda s: (s, 0, 0)),
        ],
        out_shape=[
            jax.ShapeDtypeStruct((S, N_TOK, D_MODEL), f32),
            jax.ShapeDtypeStruct((S, N_TOK, D_MODEL), f32),
            jax.ShapeDtypeStruct((S, N_TOK, D_MODEL), f32),
            jax.ShapeDtypeStruct((S, N_TOK, 768), f32),
        ],
    )(x, pos, Wq, Wk, Wv, W_rope, rmsw, dirsT, freqs, cinP, coutP)

    mask, rank = pl.pallas_call(
        _topk_kernel,
        grid=(S,),
        in_specs=[pl.BlockSpec((1, N_TOK, 768), lambda s: (s, 0, 0))],
        out_specs=[pl.BlockSpec((1, N_TOK, 768), lambda s: (s, 0, 0)),
                   pl.BlockSpec((1, N_TOK, 768), lambda s: (s, 0, 0))],
        out_shape=[jax.ShapeDtypeStruct((S, N_TOK, 768), f32),
                   jax.ShapeDtypeStruct((S, N_TOK, 768), f32)],
    )(score)

    o_pre = pl.pallas_call(
        _attn_kernel,
        grid=(S, N_HEADS),
        in_specs=[
            pl.BlockSpec((1, N_TOK, HEAD_DIM), lambda s, h: (s, 0, h)),
            pl.BlockSpec((1, N_TOK, HEAD_DIM), lambda s, h: (s, 0, h)),
            pl.BlockSpec((1, N_TOK, HEAD_DIM), lambda s, h: (s, 0, h)),
            pl.BlockSpec((1, N_TOK, N_CLUSTERS), lambda s, h: (s, 0, h)),
            pl.BlockSpec((1, N_TOK, N_CLUSTERS), lambda s, h: (s, 0, h)),
            pl.BlockSpec((1, N_TOK, N_CLUSTERS),
                         lambda s, h: (s, 0, N_HEADS + h)),
            pl.BlockSpec((1, N_TOK, N_CLUSTERS),
                         lambda s, h: (s, 0, N_HEADS + h)),
            pl.BlockSpec((1, HEAD_DIM), lambda s, h: (h, 0)),
        ],
        out_specs=pl.BlockSpec((1, N_TOK, HEAD_DIM), lambda s, h: (s, 0, h)),
        out_shape=jax.ShapeDtypeStruct((S, N_TOK, D_MODEL), f32),
    )(q, k, v, mask, rank, mask, rank, null_h)

    y = pl.pallas_call(
        _wo_kernel,
        grid=(S,),
        in_specs=[pl.BlockSpec((1, N_TOK, D_MODEL), lambda s: (s, 0, 0)),
                  full((D_MODEL, D_MODEL))],
        out_specs=pl.BlockSpec((1, N_TOK, D_MODEL), lambda s: (s, 0, 0)),
        out_shape=jax.ShapeDtypeStruct((S, N_TOK, D_MODEL), f32),
    )(o_pre, Wo)
    return y


# speedup vs baseline: 3.7386x; 3.7386x over previous
import jax
import jax.numpy as jnp
from jax.experimental import pallas as pl

D_MODEL = 768
N_HEADS = 12
HEAD_DIM = 64
N_CLUSTERS = 32
CLUSTER_K = 64
N_ROPE = 32
N_TOK = 2048
TN = 512

_PREC = jax.lax.Precision.HIGHEST
_LIM = 0x3FA00000


def _hi_lo(x):
    hi = x.astype(jnp.bfloat16)
    lo = (x - hi.astype(jnp.float32)).astype(jnp.bfloat16)
    return hi, lo


def _dot(a, b, dims, prec=None):
    return jax.lax.dot_general(a, b, (dims, ((), ())), precision=prec,
                               preferred_element_type=jnp.float32)


def _qkv_kernel(x_ref, pos_ref, wq_ref, wk_ref, wv_ref, wr_ref, rmsw_ref,
                dirs_ref, freqs_ref, q_ref, k_ref, v_ref):
    xt = x_ref[0]
    post = pos_ref[0]
    xs = xt * jax.lax.rsqrt(jnp.mean(xt * xt, axis=0, keepdims=True) + 1e-6)
    xs = xs * rmsw_ref[...]
    angles = _dot(dirs_ref[...], post, ((1,), (0,)), _PREC) * freqs_ref[...]
    embt = jnp.concatenate([jnp.sin(angles), jnp.cos(angles)], axis=0)
    ropet = _dot(wr_ref[...], embt, ((0,), (0,)), _PREC)
    q_ref[0] = _dot(wq_ref[...], xs, ((0,), (0,)), _PREC) + ropet
    k_ref[0] = _dot(wk_ref[...], xs, ((0,), (0,)), _PREC) + ropet
    v_ref[0] = _dot(wv_ref[...], xs, ((0,), (0,)), _PREC)


def _score_kernel(q_ref, k_ref, pos_ref, cin_ref, cout_ref,
                  sin_ref, sout_ref):
    post = pos_ref[0]

    def _norm(c):
        return c / jnp.sqrt(jnp.clip(jnp.sum(c * c, axis=-1, keepdims=True),
                                     1e-12, None))
    cin = _norm(cin_ref[...])
    cout = _norm(cout_ref[...])
    p_norm2 = jnp.sum(post * post, axis=0, keepdims=True)
    qt = q_ref[0]
    kt = k_ref[0]
    cn_q = jnp.sqrt(jnp.clip(jnp.sum(qt * qt, axis=0, keepdims=True)
                             + p_norm2, 1e-12, None))
    cn_k = jnp.sqrt(jnp.clip(jnp.sum(kt * kt, axis=0, keepdims=True)
                             + p_norm2, 1e-12, None))
    in_s = (_dot(cin[:, :HEAD_DIM], qt, ((1,), (0,)), _PREC)
            + _dot(cin[:, HEAD_DIM:], post, ((1,), (0,)), _PREC))
    out_s = (_dot(cout[:, :HEAD_DIM], kt, ((1,), (0,)), _PREC)
             + _dot(cout[:, HEAD_DIM:], post, ((1,), (0,)), _PREC))
    sin_ref[0] = in_s / cn_q
    sout_ref[0] = out_s / cn_k


def _topk_kernel(score_ref, mask_ref, rank_ref):
    s = score_ref[0]
    b = jax.lax.bitcast_convert_type(s, jnp.int32)
    m = jnp.where(b < 0, jnp.bitwise_xor(jnp.bitwise_not(b),
                                         jnp.int32(-2147483648)), b)
    nrows = s.shape[0]
    lo = jnp.full((nrows, 1), -_LIM - 1, dtype=jnp.int32)
    hi = jnp.full((nrows, 1), _LIM, dtype=jnp.int32)
    for _ in range(31):
        mid = lo + (hi - lo + 1) // 2
        cnt = jnp.sum((m >= mid).astype(jnp.int32), axis=1, keepdims=True)
        ge = cnt >= CLUSTER_K
        lo = jnp.where(ge, mid, lo)
        hi = jnp.where(ge, hi, mid - 1)
    vstar = lo
    gt = m > vstar
    eq = m == vstar
    cnt_gt = jnp.sum(gt.astype(jnp.int32), axis=1, keepdims=True)
    r = CLUSTER_K - cnt_gt
    idx = jax.lax.broadcasted_iota(jnp.int32, m.shape, 1)
    lo2 = jnp.zeros((nrows, 1), dtype=jnp.int32)
    hi2 = jnp.full((nrows, 1), N_TOK - 1, dtype=jnp.int32)
    for _ in range(11):
        mid2 = (lo2 + hi2) // 2
        c2 = jnp.sum((eq & (idx <= mid2)).astype(jnp.int32),
                     axis=1, keepdims=True)
        geq = c2 >= r
        hi2 = jnp.where(geq, mid2, hi2)
        lo2 = jnp.where(geq, lo2, mid2 + 1)
    sel = gt | (eq & (idx <= hi2))
    maskf = sel.astype(jnp.float32)
    utri = (jax.lax.broadcasted_iota(jnp.int32, (N_TOK, N_TOK), 0)
            < jax.lax.broadcasted_iota(jnp.int32, (N_TOK, N_TOK), 1)
            ).astype(jnp.bfloat16)
    rank = _dot(maskf.astype(jnp.bfloat16), utri, ((1,), (0,)))
    mask_ref[0] = maskf
    rank_ref[0] = rank


def _attn_kernel(q_ref, k_ref, v_ref, mq_ref, rq_ref, mkv_ref, rkv_ref,
                 null_ref, o_ref):
    mq = mq_ref[0]
    rq = rq_ref[0]
    mkv = mkv_ref[0]
    rkv = rkv_ref[0]
    iota64 = jax.lax.broadcasted_iota(jnp.int32, (CLUSTER_K, 1), 0)

    def build_pt(mask, rank):
        blocks = []
        for c in range(N_CLUSTERS):
            mc = mask[c:c + 1, :]
            rc = rank[c:c + 1, :].astype(jnp.int32)
            blocks.append(((rc == iota64) & (mc > 0.5)).astype(jnp.bfloat16))
        return jnp.concatenate(blocks, axis=0)

    ptq = build_pt(mq, rq)
    ptkv = build_pt(mkv, rkv)

    def gather(pt, xt):
        hi, lo = _hi_lo(xt)
        return (_dot(hi, pt, ((1,), (1,))) + _dot(lo, pt, ((1,), (1,))))

    q_c = gather(ptq, q_ref[0])
    k_c = gather(ptkv, k_ref[0])
    v_c = gather(ptkv, v_ref[0])

    qh_, ql_ = _hi_lo(q_c)
    kh_, kl_ = _hi_lo(k_c)
    lg = (_dot(qh_, kh_, ((0,), (0,))) + _dot(qh_, kl_, ((0,), (0,)))
          + _dot(ql_, kh_, ((0,), (0,)))) * (1.0 / 8.0)
    null = null_ref[0]
    nh_, nl_ = _hi_lo(null)
    ln = (_dot(qh_, nh_, ((0,), (1,))) + _dot(qh_, nl_, ((0,), (1,)))
          + _dot(ql_, nh_, ((0,), (1,)))) * (1.0 / 8.0)

    dblocks = [lg[c * CLUSTER_K:(c + 1) * CLUSTER_K,
                  c * CLUSTER_K:(c + 1) * CLUSTER_K]
               for c in range(N_CLUSTERS)]
    lgc = jnp.concatenate(dblocks, axis=0)
    mx = jnp.maximum(jnp.max(lgc, axis=1, keepdims=True), ln)
    e = jnp.exp(lgc - mx)
    en = jnp.exp(ln - mx)
    denom = jnp.sum(e, axis=1, keepdims=True) + en
    attnc = e / denom
    attn_null = en / denom

    ah, al = _hi_lo(attnc)

    def assemble(blockmat):
        rows = []
        for c in range(N_CLUSTERS):
            blk = blockmat[c * CLUSTER_K:(c + 1) * CLUSTER_K, :]
            pieces = []
            if c > 0:
                pieces.append(jnp.zeros((CLUSTER_K, c * CLUSTER_K),
                                        dtype=blk.dtype))
            pieces.append(blk)
            if c < N_CLUSTERS - 1:
                pieces.append(jnp.zeros(
                    (CLUSTER_K, (N_CLUSTERS - 1 - c) * CLUSTER_K),
                    dtype=blk.dtype))
            rows.append(jnp.concatenate(pieces, axis=1)
                        if len(pieces) > 1 else pieces[0])
        return jnp.concatenate(rows, axis=0)

    afh = assemble(ah)
    afl = assemble(al)
    vh_, vl_ = _hi_lo(v_c)
    out_c = (_dot(vh_, afh, ((1,), (1,))) + _dot(vl_, afh, ((1,), (1,)))
             + _dot(vh_, afl, ((1,), (1,))))
    out_c = out_c + _dot(null, attn_null, ((0,), (1,)), _PREC)
    och, ocl = _hi_lo(out_c)
    acc = _dot(och, ptq, ((1,), (0,))) + _dot(ocl, ptq, ((1,), (0,)))
    cnt = jnp.sum(mq, axis=0, keepdims=True)
    o_ref[0] = acc / jnp.maximum(cnt, 1.0)


def _wo_kernel(x_ref, wo_ref, y_ref):
    y_ref[0] = _dot(x_ref[0], wo_ref[...], ((0,), (0,)), _PREC)


@jax.jit
def kernel(x, unit_point_positions, Wq, Wk, Wv, Wo, W_rope, rms_w, null_vec,
           input_centroids, output_centroids, rope_dirs, rope_freqs):
    S = x.shape[0]
    f32 = jnp.float32
    xt = x.transpose(0, 2, 1)
    post = unit_point_positions.transpose(0, 2, 1)
    cinP = input_centroids.reshape(N_HEADS * N_CLUSTERS, HEAD_DIM + 3)
    coutP = output_centroids.reshape(N_HEADS * N_CLUSTERS, HEAD_DIM + 3)
    freqs = rope_freqs.reshape(N_ROPE, 1)
    rmsw = rms_w.reshape(D_MODEL, 1)
    null_h = null_vec.reshape(N_HEADS, 1, HEAD_DIM)

    def full(shape):
        return pl.BlockSpec(shape, lambda *a, _n=len(shape): (0,) * _n)

    NT = N_TOK // TN
    q, k, v = pl.pallas_call(
        _qkv_kernel,
        grid=(S, NT),
        in_specs=[
            pl.BlockSpec((1, D_MODEL, TN), lambda s, t: (s, 0, t)),
            pl.BlockSpec((1, 3, TN), lambda s, t: (s, 0, t)),
            full((D_MODEL, D_MODEL)), full((D_MODEL, D_MODEL)),
            full((D_MODEL, D_MODEL)), full((2 * N_ROPE, D_MODEL)),
            full((D_MODEL, 1)), full((N_ROPE, 3)), full((N_ROPE, 1)),
        ],
        out_specs=[
            pl.BlockSpec((1, D_MODEL, TN), lambda s, t: (s, 0, t)),
            pl.BlockSpec((1, D_MODEL, TN), lambda s, t: (s, 0, t)),
            pl.BlockSpec((1, D_MODEL, TN), lambda s, t: (s, 0, t)),
        ],
        out_shape=[jax.ShapeDtypeStruct((S, D_MODEL, N_TOK), f32)] * 3,
    )(xt, post, Wq, Wk, Wv, W_rope, rmsw, rope_dirs, freqs)

    in_score, out_score = pl.pallas_call(
        _score_kernel,
        grid=(S, N_HEADS),
        in_specs=[
            pl.BlockSpec((1, HEAD_DIM, N_TOK), lambda s, h: (s, h, 0)),
            pl.BlockSpec((1, HEAD_DIM, N_TOK), lambda s, h: (s, h, 0)),
            pl.BlockSpec((1, 3, N_TOK), lambda s, h: (s, 0, 0)),
            pl.BlockSpec((N_CLUSTERS, HEAD_DIM + 3), lambda s, h: (h, 0)),
            pl.BlockSpec((N_CLUSTERS, HEAD_DIM + 3), lambda s, h: (h, 0)),
        ],
        out_specs=[
            pl.BlockSpec((1, N_CLUSTERS, N_TOK), lambda s, h: (s, h, 0)),
            pl.BlockSpec((1, N_CLUSTERS, N_TOK), lambda s, h: (s, h, 0)),
        ],
        out_shape=[
            jax.ShapeDtypeStruct((S, N_HEADS * N_CLUSTERS, N_TOK), f32),
            jax.ShapeDtypeStruct((S, N_HEADS * N_CLUSTERS, N_TOK), f32),
        ],
    )(q, k, post, cinP, coutP)

    score = jnp.concatenate([in_score, out_score], axis=1)

    mask, rank = pl.pallas_call(
        _topk_kernel,
        grid=(S, 2),
        in_specs=[pl.BlockSpec((1, 384, N_TOK), lambda s, t: (s, t, 0))],
        out_specs=[pl.BlockSpec((1, 384, N_TOK), lambda s, t: (s, t, 0)),
                   pl.BlockSpec((1, 384, N_TOK), lambda s, t: (s, t, 0))],
        out_shape=[jax.ShapeDtypeStruct((S, 768, N_TOK), f32),
                   jax.ShapeDtypeStruct((S, 768, N_TOK), f32)],
    )(score)

    o_pre = pl.pallas_call(
        _attn_kernel,
        grid=(S, N_HEADS),
        in_specs=[
            pl.BlockSpec((1, HEAD_DIM, N_TOK), lambda s, h: (s, h, 0)),
            pl.BlockSpec((1, HEAD_DIM, N_TOK), lambda s, h: (s, h, 0)),
            pl.BlockSpec((1, HEAD_DIM, N_TOK), lambda s, h: (s, h, 0)),
            pl.BlockSpec((1, N_CLUSTERS, N_TOK), lambda s, h: (s, h, 0)),
            pl.BlockSpec((1, N_CLUSTERS, N_TOK), lambda s, h: (s, h, 0)),
            pl.BlockSpec((1, N_CLUSTERS, N_TOK),
                         lambda s, h: (s, N_HEADS + h, 0)),
            pl.BlockSpec((1, N_CLUSTERS, N_TOK),
                         lambda s, h: (s, N_HEADS + h, 0)),
            pl.BlockSpec((1, 1, HEAD_DIM), lambda s, h: (h, 0, 0)),
        ],
        out_specs=pl.BlockSpec((1, HEAD_DIM, N_TOK), lambda s, h: (s, h, 0)),
        out_shape=jax.ShapeDtypeStruct((S, D_MODEL, N_TOK), f32),
    )(q, k, v, mask, rank, mask, rank, null_h)

    y = pl.pallas_call(
        _wo_kernel,
        grid=(S, NT),
        in_specs=[pl.BlockSpec((1, D_MODEL, TN), lambda s, t: (s, 0, t)),
                  full((D_MODEL, D_MODEL))],
        out_specs=pl.BlockSpec((1, TN, D_MODEL), lambda s, t: (s, t, 0)),
        out_shape=jax.ShapeDtypeStruct((S, N_TOK, D_MODEL), f32),
    )(o_pre, Wo)
    return y


# speedup vs baseline: 4.1844x; 1.1192x over previous
import jax
import jax.numpy as jnp
from jax.experimental import pallas as pl

D_MODEL = 768
N_HEADS = 12
HEAD_DIM = 64
N_CLUSTERS = 32
CLUSTER_K = 64
N_ROPE = 32
N_TOK = 2048
TN = 512

_PREC = jax.lax.Precision.HIGHEST
_LIM = 0x3FA00000


def _hi_lo(x):
    hi = x.astype(jnp.bfloat16)
    lo = (x - hi.astype(jnp.float32)).astype(jnp.bfloat16)
    return hi, lo


def _dot(a, b, dims, prec=None):
    return jax.lax.dot_general(a, b, (dims, ((), ())), precision=prec,
                               preferred_element_type=jnp.float32)


def _qkv_kernel(x_ref, pos_ref, wq_ref, wk_ref, wv_ref, wr_ref, rmsw_ref,
                dirs_ref, freqs_ref, q_ref, k_ref, v_ref):
    xt = x_ref[0]
    post = pos_ref[0]
    xs = xt * jax.lax.rsqrt(jnp.mean(xt * xt, axis=0, keepdims=True) + 1e-6)
    xs = xs * rmsw_ref[...]
    angles = _dot(dirs_ref[...], post, ((1,), (0,)), _PREC) * freqs_ref[...]
    embt = jnp.concatenate([jnp.sin(angles), jnp.cos(angles)], axis=0)
    ropet = _dot(wr_ref[...], embt, ((0,), (0,)), _PREC)
    q_ref[0] = _dot(wq_ref[...], xs, ((0,), (0,)), _PREC) + ropet
    k_ref[0] = _dot(wk_ref[...], xs, ((0,), (0,)), _PREC) + ropet
    v_ref[0] = _dot(wv_ref[...], xs, ((0,), (0,)), _PREC)


def _score_kernel(q_ref, k_ref, pos_ref, cin_ref, cout_ref,
                  sin_ref, sout_ref):
    post = pos_ref[0]

    def _norm(c):
        return c / jnp.sqrt(jnp.clip(jnp.sum(c * c, axis=-1, keepdims=True),
                                     1e-12, None))
    cin = _norm(cin_ref[...])
    cout = _norm(cout_ref[...])
    p_norm2 = jnp.sum(post * post, axis=0, keepdims=True)
    qt = q_ref[0]
    kt = k_ref[0]
    cn_q = jnp.sqrt(jnp.clip(jnp.sum(qt * qt, axis=0, keepdims=True)
                             + p_norm2, 1e-12, None))
    cn_k = jnp.sqrt(jnp.clip(jnp.sum(kt * kt, axis=0, keepdims=True)
                             + p_norm2, 1e-12, None))
    in_s = (_dot(cin[:, :HEAD_DIM], qt, ((1,), (0,)), _PREC)
            + _dot(cin[:, HEAD_DIM:], post, ((1,), (0,)), _PREC))
    out_s = (_dot(cout[:, :HEAD_DIM], kt, ((1,), (0,)), _PREC)
             + _dot(cout[:, HEAD_DIM:], post, ((1,), (0,)), _PREC))
    sin_ref[0] = in_s / cn_q
    sout_ref[0] = out_s / cn_k


def _topk_kernel(score_ref, code_ref):
    s = score_ref[0]
    b = jax.lax.bitcast_convert_type(s, jnp.int32)
    m = jnp.where(b < 0, jnp.bitwise_xor(jnp.bitwise_not(b),
                                         jnp.int32(-2147483648)), b)
    nrows = s.shape[0]
    lo = jnp.full((nrows, 1), -_LIM - 1, dtype=jnp.int32)
    hi = jnp.full((nrows, 1), _LIM, dtype=jnp.int32)
    for _ in range(31):
        mid = lo + (hi - lo + 1) // 2
        cnt = jnp.sum((m >= mid).astype(jnp.int32), axis=1, keepdims=True)
        ge = cnt >= CLUSTER_K
        lo = jnp.where(ge, mid, lo)
        hi = jnp.where(ge, hi, mid - 1)
    vstar = lo
    gt = m > vstar
    eq = m == vstar
    cnt_gt = jnp.sum(gt.astype(jnp.int32), axis=1, keepdims=True)
    r = CLUSTER_K - cnt_gt
    idx = jax.lax.broadcasted_iota(jnp.int32, m.shape, 1)
    lo2 = jnp.zeros((nrows, 1), dtype=jnp.int32)
    hi2 = jnp.full((nrows, 1), N_TOK - 1, dtype=jnp.int32)
    for _ in range(11):
        mid2 = (lo2 + hi2) // 2
        c2 = jnp.sum((eq & (idx <= mid2)).astype(jnp.int32),
                     axis=1, keepdims=True)
        geq = c2 >= r
        hi2 = jnp.where(geq, mid2, hi2)
        lo2 = jnp.where(geq, lo2, mid2 + 1)
    sel = gt | (eq & (idx <= hi2))
    maskf = sel.astype(jnp.float32)
    utri = (jax.lax.broadcasted_iota(jnp.int32, (N_TOK, N_TOK), 0)
            < jax.lax.broadcasted_iota(jnp.int32, (N_TOK, N_TOK), 1)
            ).astype(jnp.bfloat16)
    rank = _dot(maskf.astype(jnp.bfloat16), utri, ((1,), (0,)))
    code_ref[0] = rank + (1.0 - maskf) * 128.0


def _attn_kernel(q_ref, k_ref, v_ref, cq_ref, ckv_ref, null_ref, o_ref):
    cq = cq_ref[0]
    ckv = ckv_ref[0]
    CK = N_CLUSTERS * CLUSTER_K
    et = (jax.lax.broadcasted_iota(jnp.int32, (CK, N_CLUSTERS), 0) // CLUSTER_K
          == jax.lax.broadcasted_iota(jnp.int32, (CK, N_CLUSTERS), 1)
          ).astype(jnp.bfloat16)
    kmod = jnp.bitwise_and(
        jax.lax.broadcasted_iota(jnp.int32, (CK, 1), 0),
        jnp.int32(CLUSTER_K - 1)).astype(jnp.float32)

    def build_pt(code):
        rep = _dot(et, code.astype(jnp.bfloat16), ((1,), (0,)))
        return (rep == kmod).astype(jnp.bfloat16)

    ptq = build_pt(cq)
    ptkv = build_pt(ckv)

    def gather2(pt, xt):
        hi, lo = _hi_lo(xt)
        return (_dot(hi, pt, ((1,), (1,))) + _dot(lo, pt, ((1,), (1,))))

    q_c = gather2(ptq, q_ref[0])
    k_c = gather2(ptkv, k_ref[0])
    v_c = _dot(v_ref[0].astype(jnp.bfloat16), ptkv, ((1,), (1,)))

    qh_, ql_ = _hi_lo(q_c)
    kh_, kl_ = _hi_lo(k_c)
    lg = (_dot(qh_, kh_, ((0,), (0,))) + _dot(qh_, kl_, ((0,), (0,)))
          + _dot(ql_, kh_, ((0,), (0,)))) * (1.0 / 8.0)
    null = null_ref[0]
    nh_, nl_ = _hi_lo(null)
    ln = (_dot(qh_, nh_, ((0,), (1,))) + _dot(qh_, nl_, ((0,), (1,)))
          + _dot(ql_, nh_, ((0,), (1,)))) * (1.0 / 8.0)

    dblocks = [lg[c * CLUSTER_K:(c + 1) * CLUSTER_K,
                  c * CLUSTER_K:(c + 1) * CLUSTER_K]
               for c in range(N_CLUSTERS)]
    lgc = jnp.concatenate(dblocks, axis=0)
    mx = jnp.maximum(jnp.max(lgc, axis=1, keepdims=True), ln)
    e = jnp.exp(lgc - mx)
    en = jnp.exp(ln - mx)
    denom = jnp.sum(e, axis=1, keepdims=True) + en
    attnc = e / denom
    attn_null = en / denom

    ah, al = _hi_lo(attnc)

    def assemble(blockmat):
        rows = []
        for c in range(N_CLUSTERS):
            blk = blockmat[c * CLUSTER_K:(c + 1) * CLUSTER_K, :]
            pieces = []
            if c > 0:
                pieces.append(jnp.zeros((CLUSTER_K, c * CLUSTER_K),
                                        dtype=blk.dtype))
            pieces.append(blk)
            if c < N_CLUSTERS - 1:
                pieces.append(jnp.zeros(
                    (CLUSTER_K, (N_CLUSTERS - 1 - c) * CLUSTER_K),
                    dtype=blk.dtype))
            rows.append(jnp.concatenate(pieces, axis=1)
                        if len(pieces) > 1 else pieces[0])
        return jnp.concatenate(rows, axis=0)

    afh = assemble(ah)
    afl = assemble(al)
    vh_ = v_c.astype(jnp.bfloat16)
    out_c = (_dot(vh_, afh, ((1,), (1,))) + _dot(vh_, afl, ((1,), (1,))))
    out_c = out_c + _dot(null, attn_null, ((0,), (1,)), _PREC)
    acc = _dot(out_c.astype(jnp.bfloat16), ptq, ((1,), (0,)))
    cnt = jnp.sum((cq < 64.0).astype(jnp.float32), axis=0, keepdims=True)
    o_ref[0] = acc / jnp.maximum(cnt, 1.0)


def _wo_kernel(x_ref, wo_ref, y_ref):
    y_ref[0] = _dot(x_ref[0], wo_ref[...], ((0,), (0,)), _PREC)


@jax.jit
def kernel(x, unit_point_positions, Wq, Wk, Wv, Wo, W_rope, rms_w, null_vec,
           input_centroids, output_centroids, rope_dirs, rope_freqs):
    S = x.shape[0]
    f32 = jnp.float32
    xt = x.transpose(0, 2, 1)
    post = unit_point_positions.transpose(0, 2, 1)
    cinP = input_centroids.reshape(N_HEADS * N_CLUSTERS, HEAD_DIM + 3)
    coutP = output_centroids.reshape(N_HEADS * N_CLUSTERS, HEAD_DIM + 3)
    freqs = rope_freqs.reshape(N_ROPE, 1)
    rmsw = rms_w.reshape(D_MODEL, 1)
    null_h = null_vec.reshape(N_HEADS, 1, HEAD_DIM)

    def full(shape):
        return pl.BlockSpec(shape, lambda *a, _n=len(shape): (0,) * _n)

    NT = N_TOK // TN
    q, k, v = pl.pallas_call(
        _qkv_kernel,
        grid=(S, NT),
        in_specs=[
            pl.BlockSpec((1, D_MODEL, TN), lambda s, t: (s, 0, t)),
            pl.BlockSpec((1, 3, TN), lambda s, t: (s, 0, t)),
            full((D_MODEL, D_MODEL)), full((D_MODEL, D_MODEL)),
            full((D_MODEL, D_MODEL)), full((2 * N_ROPE, D_MODEL)),
            full((D_MODEL, 1)), full((N_ROPE, 3)), full((N_ROPE, 1)),
        ],
        out_specs=[
            pl.BlockSpec((1, D_MODEL, TN), lambda s, t: (s, 0, t)),
            pl.BlockSpec((1, D_MODEL, TN), lambda s, t: (s, 0, t)),
            pl.BlockSpec((1, D_MODEL, TN), lambda s, t: (s, 0, t)),
        ],
        out_shape=[jax.ShapeDtypeStruct((S, D_MODEL, N_TOK), f32)] * 3,
    )(xt, post, Wq, Wk, Wv, W_rope, rmsw, rope_dirs, freqs)

    in_score, out_score = pl.pallas_call(
        _score_kernel,
        grid=(S, N_HEADS),
        in_specs=[
            pl.BlockSpec((1, HEAD_DIM, N_TOK), lambda s, h: (s, h, 0)),
            pl.BlockSpec((1, HEAD_DIM, N_TOK), lambda s, h: (s, h, 0)),
            pl.BlockSpec((1, 3, N_TOK), lambda s, h: (s, 0, 0)),
            pl.BlockSpec((N_CLUSTERS, HEAD_DIM + 3), lambda s, h: (h, 0)),
            pl.BlockSpec((N_CLUSTERS, HEAD_DIM + 3), lambda s, h: (h, 0)),
        ],
        out_specs=[
            pl.BlockSpec((1, N_CLUSTERS, N_TOK), lambda s, h: (s, h, 0)),
            pl.BlockSpec((1, N_CLUSTERS, N_TOK), lambda s, h: (s, h, 0)),
        ],
        out_shape=[
            jax.ShapeDtypeStruct((S, N_HEADS * N_CLUSTERS, N_TOK), f32),
            jax.ShapeDtypeStruct((S, N_HEADS * N_CLUSTERS, N_TOK), f32),
        ],
    )(q, k, post, cinP, coutP)

    score = jnp.concatenate([in_score, out_score], axis=1)

    code = pl.pallas_call(
        _topk_kernel,
        grid=(S, 2),
        in_specs=[pl.BlockSpec((1, 384, N_TOK), lambda s, t: (s, t, 0))],
        out_specs=pl.BlockSpec((1, 384, N_TOK), lambda s, t: (s, t, 0)),
        out_shape=jax.ShapeDtypeStruct((S, 768, N_TOK), f32),
    )(score)

    o_pre = pl.pallas_call(
        _attn_kernel,
        grid=(S, N_HEADS),
        in_specs=[
            pl.BlockSpec((1, HEAD_DIM, N_TOK), lambda s, h: (s, h, 0)),
            pl.BlockSpec((1, HEAD_DIM, N_TOK), lambda s, h: (s, h, 0)),
            pl.BlockSpec((1, HEAD_DIM, N_TOK), lambda s, h: (s, h, 0)),
            pl.BlockSpec((1, N_CLUSTERS, N_TOK), lambda s, h: (s, h, 0)),
            pl.BlockSpec((1, N_CLUSTERS, N_TOK),
                         lambda s, h: (s, N_HEADS + h, 0)),
            pl.BlockSpec((1, 1, HEAD_DIM), lambda s, h: (h, 0, 0)),
        ],
        out_specs=pl.BlockSpec((1, HEAD_DIM, N_TOK), lambda s, h: (s, h, 0)),
        out_shape=jax.ShapeDtypeStruct((S, D_MODEL, N_TOK), f32),
    )(q, k, v, code, code, null_h)

    y = pl.pallas_call(
        _wo_kernel,
        grid=(S, NT),
        in_specs=[pl.BlockSpec((1, D_MODEL, TN), lambda s, t: (s, 0, t)),
                  full((D_MODEL, D_MODEL))],
        out_specs=pl.BlockSpec((1, TN, D_MODEL), lambda s, t: (s, t, 0)),
        out_shape=jax.ShapeDtypeStruct((S, N_TOK, D_MODEL), f32),
    )(o_pre, Wo)
    return y
